# baseline (speedup 1.0000x reference)
import jax
import jax.numpy as jnp
from jax import lax
from jax.experimental import pallas as pl
from jax.experimental.pallas import tpu as pltpu
from jax.experimental.pallas import tpu_sc as plsc

_EMBED = 32
_BATCH = 16384
_HIST = 50
_TBL_ROWS = 101

_NW = 32
_CHUNK = 1280
_NBUF = 2
_L = 16


def _gather_kernel(idx_hbm, table_hbm, out_hbm,
                   tbl_v, idx_v, out_v, tsem, isem, osem):
    nflat = _BATCH * _HIST
    b_per_w = nflat // _NW
    steps = b_per_w // _CHUNK
    wid = lax.axis_index("s") * 2 + lax.axis_index("c")
    row0 = wid * b_per_w

    def idx_copy(c, b):
        return pltpu.make_async_copy(
            idx_hbm.at[pl.ds(row0 + c * _CHUNK, _CHUNK)],
            idx_v.at[b], isem.at[b])

    def writeout_copy(c, b):
        return pltpu.make_async_copy(
            out_v.at[b],
            out_hbm.at[pl.ds((row0 + c * _CHUNK) * _EMBED, _CHUNK * _EMBED)],
            osem.at[b])

    pltpu.make_async_copy(table_hbm, tbl_v, tsem).start()
    pltpu.make_async_copy(table_hbm, tbl_v, tsem).wait()

    viota = lax.iota(jnp.int32, _L)
    viota_e = viota * _EMBED

    def compute_chunk(b):
        @pl.loop(0, _CHUNK // _L)
        def _(k):
            r0 = k * _L
            idxv = idx_v[b, pl.ds(r0, _L)]
            gbase = idxv * _EMBED
            pbase = viota_e + r0 * _EMBED
            for col in range(_EMBED):
                v = plsc.load_gather(tbl_v, [gbase + col])
                plsc.store_scatter(out_v.at[b], [pbase + col], v)

    for b in range(_NBUF):
        idx_copy(b, b).start()

    @pl.loop(0, steps, step=_NBUF)
    def _(g):
        for b in range(_NBUF):
            c = g + b
            idx_copy(c, b).wait()

            @pl.when(c >= _NBUF)
            def _():
                writeout_copy(c - _NBUF, b).wait()

            compute_chunk(b)
            writeout_copy(c, b).start()

            @pl.when(c + _NBUF < steps)
            def _():
                idx_copy(c + _NBUF, b).start()

    for b in range(_NBUF):
        writeout_copy(steps - _NBUF + b, b).wait()


def kernel(indices, table):
    nflat = indices.shape[0] * indices.shape[1]
    idx_flat = indices.reshape(nflat)
    tbl_flat = table.reshape(table.shape[0] * table.shape[1])
    mesh = plsc.VectorSubcoreMesh(core_axis_name="c", subcore_axis_name="s")
    call = pl.kernel(
        _gather_kernel,
        mesh=mesh,
        out_type=jax.ShapeDtypeStruct((nflat * _EMBED,), jnp.float32),
        scratch_types=[
            pltpu.VMEM((_TBL_ROWS * _EMBED,), jnp.float32),
            pltpu.VMEM((_NBUF, _CHUNK), jnp.int32),
            pltpu.VMEM((_NBUF, _CHUNK * _EMBED), jnp.float32),
            pltpu.SemaphoreType.DMA,
            pltpu.SemaphoreType.DMA((_NBUF,)),
            pltpu.SemaphoreType.DMA((_NBUF,)),
        ],
        compiler_params=pltpu.CompilerParams(use_tc_tiling_on_sc=False,
                                             needs_layout_passes=False),
    )
    out = call(idx_flat, tbl_flat)
    return out.reshape(indices.shape[0], _HIST * _EMBED)


# speedup vs baseline: 1.3166x; 1.3166x over previous
import jax
import jax.numpy as jnp
from jax import lax
from jax.experimental import pallas as pl
from jax.experimental.pallas import tpu as pltpu
from jax.experimental.pallas import tpu_sc as plsc

_EMBED = 32
_BATCH = 16384
_HIST = 50
_TBL_ROWS = 101

_NW = 32
_CHUNK = 1280
_NBUF = 2
_L = 16


def _gather_kernel(idx_hbm, table_hbm, out_hbm,
                   tbl_v, idx_v, out_v, tsem, isem, osem):
    nflat = _BATCH * _HIST
    b_per_w = nflat // _NW
    steps = b_per_w // _CHUNK
    wid = lax.axis_index("s") * 2 + lax.axis_index("c")
    row0 = wid * b_per_w

    def idx_copy(c, b):
        return pltpu.make_async_copy(
            idx_hbm.at[pl.ds(row0 + c * _CHUNK, _CHUNK)],
            idx_v.at[b], isem.at[b])

    def writeout_copy(c, b):
        return pltpu.make_async_copy(
            out_v.at[b],
            out_hbm.at[pl.ds((row0 + c * _CHUNK) * _EMBED, _CHUNK * _EMBED)],
            osem.at[b])

    pltpu.make_async_copy(table_hbm, tbl_v, tsem).start()
    pltpu.make_async_copy(table_hbm, tbl_v, tsem).wait()

    viota = lax.iota(jnp.int32, _L)
    viota_e = viota * _EMBED

    def compute_chunk(b):
        @plsc.parallel_loop(0, _CHUNK // _L, unroll=2)
        def _(k):
            r0 = k * _L
            idxv = idx_v[b, pl.ds(r0, _L)]
            gbase = idxv * _EMBED
            pbase = viota_e + r0 * _EMBED
            for c0 in range(0, _EMBED, 8):
                vs = [plsc.load_gather(tbl_v, [gbase + (c0 + i)])
                      for i in range(8)]
                for i in range(8):
                    plsc.store_scatter(out_v.at[b], [pbase + (c0 + i)], vs[i])

    for b in range(_NBUF):
        idx_copy(b, b).start()

    @pl.loop(0, steps, step=_NBUF)
    def _(g):
        for b in range(_NBUF):
            c = g + b
            idx_copy(c, b).wait()

            @pl.when(c >= _NBUF)
            def _():
                writeout_copy(c - _NBUF, b).wait()

            compute_chunk(b)
            writeout_copy(c, b).start()

            @pl.when(c + _NBUF < steps)
            def _():
                idx_copy(c + _NBUF, b).start()

    for b in range(_NBUF):
        writeout_copy(steps - _NBUF + b, b).wait()


def kernel(indices, table):
    nflat = indices.shape[0] * indices.shape[1]
    idx_flat = indices.reshape(nflat)
    tbl_flat = table.reshape(table.shape[0] * table.shape[1])
    mesh = plsc.VectorSubcoreMesh(core_axis_name="c", subcore_axis_name="s")
    call = pl.kernel(
        _gather_kernel,
        mesh=mesh,
        out_type=jax.ShapeDtypeStruct((nflat * _EMBED,), jnp.float32),
        scratch_types=[
            pltpu.VMEM((_TBL_ROWS * _EMBED,), jnp.float32),
            pltpu.VMEM((_NBUF, _CHUNK), jnp.int32),
            pltpu.VMEM((_NBUF, _CHUNK * _EMBED), jnp.float32),
            pltpu.SemaphoreType.DMA,
            pltpu.SemaphoreType.DMA((_NBUF,)),
            pltpu.SemaphoreType.DMA((_NBUF,)),
        ],
        compiler_params=pltpu.CompilerParams(use_tc_tiling_on_sc=False,
                                             needs_layout_passes=False),
    )
    out = call(idx_flat, tbl_flat)
    return out.reshape(indices.shape[0], _HIST * _EMBED)


# speedup vs baseline: 4.4749x; 3.3989x over previous
import jax
import jax.numpy as jnp
from jax import lax
from jax.experimental import pallas as pl
from jax.experimental.pallas import tpu as pltpu
from jax.experimental.pallas import tpu_sc as plsc

_EMBED = 32
_BATCH = 16384
_HIST = 50
_TBL_ROWS = 101

_NW = 32
_CHUNK = 1280
_NBUF = 2
_L = 16


def _gather_kernel(idx_hbm, table_hbm, out_hbm,
                   tbl_v, idx_v, out_v, tsem, isem, osem):
    nflat = _BATCH * _HIST
    b_per_w = nflat // _NW
    steps = b_per_w // _CHUNK
    wid = lax.axis_index("s") * 2 + lax.axis_index("c")
    row0 = wid * b_per_w

    def idx_copy(c, b):
        return pltpu.make_async_copy(
            idx_hbm.at[pl.ds(row0 + c * _CHUNK, _CHUNK)],
            idx_v.at[b], isem.at[b])

    def writeout_copy(c, b):
        return pltpu.make_async_copy(
            out_v.at[b],
            out_hbm.at[pl.ds((row0 + c * _CHUNK) * _EMBED, _CHUNK * _EMBED)],
            osem.at[b])

    pltpu.make_async_copy(table_hbm, tbl_v, tsem).start()
    pltpu.make_async_copy(table_hbm, tbl_v, tsem).wait()

    def compute_chunk(b):
        @plsc.parallel_loop(0, _CHUNK // _L)
        def _(k):
            r0 = k * _L
            idxv = idx_v[b, pl.ds(r0, _L)] * _EMBED
            for j in range(_L):
                base = idxv[j]
                v0 = tbl_v[pl.ds(base, _L)]
                v1 = tbl_v[pl.ds(base + _L, _L)]
                out_v[b, pl.ds((r0 + j) * _EMBED, _L)] = v0
                out_v[b, pl.ds((r0 + j) * _EMBED + _L, _L)] = v1

    for b in range(_NBUF):
        idx_copy(b, b).start()

    @pl.loop(0, steps, step=_NBUF)
    def _(g):
        for b in range(_NBUF):
            c = g + b
            idx_copy(c, b).wait()

            @pl.when(c >= _NBUF)
            def _():
                writeout_copy(c - _NBUF, b).wait()

            compute_chunk(b)
            writeout_copy(c, b).start()

            @pl.when(c + _NBUF < steps)
            def _():
                idx_copy(c + _NBUF, b).start()

    for b in range(_NBUF):
        writeout_copy(steps - _NBUF + b, b).wait()


def kernel(indices, table):
    nflat = indices.shape[0] * indices.shape[1]
    idx_flat = indices.reshape(nflat)
    tbl_flat = table.reshape(table.shape[0] * table.shape[1])
    mesh = plsc.VectorSubcoreMesh(core_axis_name="c", subcore_axis_name="s")
    call = pl.kernel(
        _gather_kernel,
        mesh=mesh,
        out_type=jax.ShapeDtypeStruct((nflat * _EMBED,), jnp.float32),
        scratch_types=[
            pltpu.VMEM((_TBL_ROWS * _EMBED,), jnp.float32),
            pltpu.VMEM((_NBUF, _CHUNK), jnp.int32),
            pltpu.VMEM((_NBUF, _CHUNK * _EMBED), jnp.float32),
            pltpu.SemaphoreType.DMA,
            pltpu.SemaphoreType.DMA((_NBUF,)),
            pltpu.SemaphoreType.DMA((_NBUF,)),
        ],
        compiler_params=pltpu.CompilerParams(use_tc_tiling_on_sc=False,
                                             needs_layout_passes=False),
    )
    out = call(idx_flat, tbl_flat)
    return out.reshape(indices.shape[0], _HIST * _EMBED)


# speedup vs baseline: 6.0828x; 1.3593x over previous
import jax
import jax.numpy as jnp
from jax import lax
from jax.experimental import pallas as pl
from jax.experimental.pallas import tpu as pltpu
from jax.experimental.pallas import tpu_sc as plsc

_EMBED = 32
_BATCH = 16384
_HIST = 50

_NW = 32
_ROWS = 16
_NBUF = 2
_L = 16


def _gather_kernel(idx_hbm, table_hbm, out_hbm,
                   tbl_v, idx_v, out_v, tsem, isem, osem):
    rows_per_w = _BATCH // _NW
    steps = rows_per_w // _ROWS
    wid = lax.axis_index("s") * 2 + lax.axis_index("c")
    row0 = wid * rows_per_w

    def writeout_copy(c, b):
        return pltpu.make_async_copy(
            out_v.at[b],
            out_hbm.at[pl.ds(row0 + c * _ROWS, _ROWS)],
            osem.at[b])

    pltpu.make_async_copy(table_hbm, tbl_v, tsem).start()
    pltpu.make_async_copy(
        idx_hbm.at[pl.ds(row0 * _HIST, rows_per_w * _HIST)], idx_v,
        isem).start()
    pltpu.make_async_copy(table_hbm, tbl_v, tsem).wait()
    pltpu.make_async_copy(
        idx_hbm.at[pl.ds(row0 * _HIST, rows_per_w * _HIST)], idx_v,
        isem).wait()

    def compute_chunk(c, b):
        @plsc.parallel_loop(0, _ROWS)
        def _(br):
            i0 = (c * _ROWS + br) * _HIST
            iv = [idx_v[pl.ds(i0 + o, _L)] * _EMBED
                  for o in (0, _L, 2 * _L, _HIST - _L)]
            for h in range(_HIST):
                q, j = divmod(h, _L)
                if q == 3:
                    q, j = 3, h - (_HIST - _L)
                base = iv[q][j]
                v0 = tbl_v[pl.ds(base, _L)]
                v1 = tbl_v[pl.ds(base + _L, _L)]
                out_v[b, br, pl.ds(h * _EMBED, _L)] = v0
                out_v[b, br, pl.ds(h * _EMBED + _L, _L)] = v1

    @pl.loop(0, steps, step=_NBUF)
    def _(g):
        for b in range(_NBUF):
            c = g + b

            @pl.when(c >= _NBUF)
            def _():
                writeout_copy(c - _NBUF, b).wait()

            compute_chunk(c, b)
            writeout_copy(c, b).start()

    for b in range(_NBUF):
        writeout_copy(steps - _NBUF + b, b).wait()


def kernel(indices, table):
    nflat = indices.shape[0] * indices.shape[1]
    idx_flat = indices.reshape(nflat)
    tbl_flat = table.reshape(table.shape[0] * table.shape[1])
    mesh = plsc.VectorSubcoreMesh(core_axis_name="c", subcore_axis_name="s")
    call = pl.kernel(
        _gather_kernel,
        mesh=mesh,
        out_type=jax.ShapeDtypeStruct((_BATCH, _HIST * _EMBED), jnp.float32),
        scratch_types=[
            pltpu.VMEM((101 * _EMBED,), jnp.float32),
            pltpu.VMEM((_BATCH // _NW * _HIST,), jnp.int32),
            pltpu.VMEM((_NBUF, _ROWS, _HIST * _EMBED), jnp.float32),
            pltpu.SemaphoreType.DMA,
            pltpu.SemaphoreType.DMA,
            pltpu.SemaphoreType.DMA((_NBUF,)),
        ],
        compiler_params=pltpu.CompilerParams(use_tc_tiling_on_sc=True,
                                             needs_layout_passes=False),
    )
    return call(idx_flat, tbl_flat)


# speedup vs baseline: 13.5902x; 2.2342x over previous
import jax
import jax.numpy as jnp
from jax import lax
from jax.experimental import pallas as pl
from jax.experimental.pallas import tpu as pltpu
from jax.experimental.pallas import tpu_sc as plsc

_EMBED = 32
_BATCH = 16384
_HIST = 50
_TSTRIDE = 33

_NW = 32
_BCOLS = 128
_HCHUNK = 5
_CROWS = _HCHUNK * _EMBED
_NBUF = 2
_L = 16


def _gather_kernel(idx_hbm, table_hbm, out_hbm,
                   tbl_v, idx_v, out_v, tsem, isem, osem):
    cols_per_w = _BATCH // _NW
    bgroups = cols_per_w // _BCOLS
    cchunks = (_HIST * _EMBED) // _CROWS
    wid = lax.axis_index("s") * 2 + lax.axis_index("c")
    col0 = wid * cols_per_w

    pltpu.make_async_copy(table_hbm, tbl_v, tsem).start()
    pltpu.make_async_copy(table_hbm, tbl_v, tsem).wait()

    viota50 = lax.iota(jnp.int32, _L) * _HIST

    pltpu.make_async_copy(
        idx_hbm.at[pl.ds(col0 * _HIST, cols_per_w * _HIST)], idx_v,
        isem).start()
    pltpu.make_async_copy(
        idx_hbm.at[pl.ds(col0 * _HIST, cols_per_w * _HIST)], idx_v,
        isem).wait()

    def writeout_copy(bg, cc, b):
        return pltpu.make_async_copy(
            out_v.at[b],
            out_hbm.at[pl.ds(cc * _CROWS, _CROWS),
                       pl.ds(col0 + bg * _BCOLS, _BCOLS)],
            osem.at[b])

    def compute_chunk(bg, cc, b):
        @plsc.parallel_loop(0, _HCHUNK * (_BCOLS // _L))
        def _(t):
            h = cc * _HCHUNK + t // (_BCOLS // _L)
            bg16 = t % (_BCOLS // _L)
            pos = viota50 + ((bg * _BCOLS + bg16 * _L) * _HIST + h)
            idxv = plsc.load_gather(idx_v, [pos])
            tbase = idxv * _TSTRIDE
            r0 = (t // (_BCOLS // _L)) * _EMBED
            for e in range(_EMBED):
                val = plsc.load_gather(tbl_v, [tbase + e])
                out_v[b, r0 + e, pl.ds(bg16 * _L, _L)] = val

    @pl.loop(0, bgroups)
    def _(bg):
        @pl.loop(0, cchunks, step=_NBUF)
        def _(g):
            for b in range(_NBUF):
                cc = g + b

                @pl.when(jnp.logical_or(bg > 0, cc >= _NBUF))
                def _():
                    writeout_copy(bg, cc, b).wait()

                compute_chunk(bg, cc, b)
                writeout_copy(bg, cc, b).start()

    for b in range(_NBUF):
        writeout_copy(bgroups - 1, cchunks - _NBUF + b, b).wait()


def kernel(indices, table):
    nflat = indices.shape[0] * indices.shape[1]
    idx_flat = indices.reshape(nflat)
    tbl_pad = jnp.pad(table, ((0, 0), (0, _TSTRIDE - _EMBED))).reshape(-1)
    tbl_pad = jnp.pad(tbl_pad, (0, 4096 - tbl_pad.shape[0]))
    mesh = plsc.VectorSubcoreMesh(core_axis_name="c", subcore_axis_name="s")
    call = pl.kernel(
        _gather_kernel,
        mesh=mesh,
        out_type=jax.ShapeDtypeStruct((_HIST * _EMBED, _BATCH), jnp.float32),
        scratch_types=[
            pltpu.VMEM((4096,), jnp.float32),
            pltpu.VMEM((_BATCH // _NW * _HIST,), jnp.int32),
            pltpu.VMEM((_NBUF, _CROWS, _BCOLS), jnp.float32),
            pltpu.SemaphoreType.DMA,
            pltpu.SemaphoreType.DMA,
            pltpu.SemaphoreType.DMA((_NBUF,)),
        ],
        compiler_params=pltpu.CompilerParams(use_tc_tiling_on_sc=True,
                                             needs_layout_passes=False),
    )
    out_t = call(idx_flat, tbl_pad)
    return out_t.T
